# baseline (speedup 1.0000x reference)
import jax
import jax.numpy as jnp
import numpy as np
from jax.experimental import pallas as pl
from jax.experimental.pallas import tpu as pltpu

_DU = 20
_DZ = 6
_DZU = _DU * _DZ
_BT = 128
_HI = jax.lax.Precision.HIGHEST
_MED = jax.lax.Precision.HIGHEST


def _g1_idx():
    idx = np.full((_DU, _DZU), 1 + 3 * _DZ, dtype=np.int32)
    for i in range(_DU):
        for t in range(3 * _DZ):
            idx[i, (6 * (i - 1) + t) % _DZU] = 1 + t
    return idx


def _g2_idx():
    idx = np.full((_DU, _DZ * 128), _DZ + 5 * _DZ * _DZ, dtype=np.int32)
    for jb in range(_DU):
        for s in range(_DZ):
            for t in range(5 * _DZ):
                idx[jb, 128 * s + (6 * (jb - 2) + t) % _DZU] = \
                    _DZ + 30 * s + t
    return idx


_G1_IDX = _g1_idx()
_G2_IDX = _g2_idx()


def _block_diag(blocks):
    g, k, n = blocks.shape
    eye = jnp.eye(g, dtype=blocks.dtype)
    bd = blocks[:, :, None, :] * eye[:, None, :, None]
    return bd.reshape(g * k, g * n)


def _expand_mid(W, b):
    wt = W.T[None].astype(jnp.float32)
    blocks = jnp.broadcast_to(wt, (_DU,) + wt.shape[1:])
    return _block_diag(blocks), jnp.tile(b, (_DU,))[None, :]


def _build_weights(params1, params2):
    (W1a, b1a), (W2a, b2a), (W3a, b3a), (W4a, b4a) = params1
    (W1b, b1b), (W2b, b2b), (W3b, b3b), (W4b, b4b) = params2

    def first(W1, b1):
        blocks = jnp.broadcast_to(W1.T[None], (_DU, 3, 16))
        eye = jnp.eye(_DU, dtype=jnp.float32)
        bd = blocks[:, :, None, :] * eye[:, None, :, None]
        w = bd.transpose(1, 0, 2, 3).reshape(3 * _DU, 16 * _DU)
        return w, jnp.tile(b1, (_DU,))[None, :]

    W1Pa, B1a = first(W1a, b1a)
    W1Pb, B1b = first(W1b, b1b)
    W2Pa, B2a = _expand_mid(W2a, b2a)
    W2Pb, B2b = _expand_mid(W2b, b2b)
    W3Pa, B3a = _expand_mid(W3a, b3a)
    W3Pb, B3b = _expand_mid(W3b, b3b)

    f1_blocks = jnp.broadcast_to(W4a[0][None, :, None], (_DU, 16, 1))
    Wf1 = _block_diag(f1_blocks)
    Bf1 = jnp.tile(b4a[0][None], (_DU,))[None, :]

    f2_blocks = jnp.broadcast_to(W4b[:_DZ].T[None], (_DU, 16, _DZ))
    Wf2 = _block_diag(f2_blocks)
    Bf2 = jnp.tile(b4b[:_DZ], (_DU,))[None, :]

    w4a_pad = jnp.concatenate([W4a.T, jnp.zeros((16, 1), jnp.float32)], 1)
    b4a_pad = jnp.concatenate([b4a, jnp.zeros((1,), jnp.float32)])
    Wg1 = w4a_pad[:, _G1_IDX].transpose(1, 0, 2)
    Bg1 = b4a_pad[_G1_IDX][:, None, :]

    w4b_pad = jnp.concatenate([W4b.T, jnp.zeros((16, 1), jnp.float32)], 1)
    b4b_pad = jnp.concatenate([b4b, jnp.zeros((1,), jnp.float32)])
    Wg2 = w4b_pad[:, _G2_IDX].transpose(1, 0, 2)
    Bg2 = b4b_pad[_G2_IDX][:, None, :]

    return (W1Pa, B1a, W2Pa, B2a, W3Pa, B3a, Wf1, Bf1, Wg1, Bg1,
            W1Pb, B1b, W2Pb, B2b, W3Pb, B3b, Wf2, Bf2, Wg2, Bg2)


def _body(x_ref,
          w1a, c1a, w2a, c2a, w3a, c3a, wf1, cf1, wg1, cg1,
          w1b, c1b, w2b, c2b, w3b, c3b, wf2, cf2, wg2, cg2,
          f1_ref, g1_ref, f2_ref, g2_ref):
    x = x_ref[...]
    xm = jnp.concatenate([x[:, -1:], x[:, :-1]], axis=1)
    xp = jnp.concatenate([x[:, 1:], x[:, :1]], axis=1)
    x3 = jnp.concatenate([xm, x, xp], axis=1)

    def mm(a, w, c):
        return jnp.dot(a.astype(jnp.bfloat16), w,
                       precision=jax.lax.Precision.DEFAULT,
                       preferred_element_type=jnp.float32) + c

    ha = jnp.maximum(mm(x3, w1a[...], c1a[...]), 0.0)
    ha = jnp.maximum(mm(ha, w2a[...], c2a[...]), 0.0)
    ha = jnp.maximum(mm(ha, w3a[...], c3a[...]), 0.0)
    f1_ref[...] = mm(ha, wf1[...], cf1[...])
    for i in range(_DU):
        g1_ref[:, i, :] = mm(
            ha[:, 16 * i:16 * (i + 1)], wg1[i], cg1[i])

    hb = jnp.maximum(mm(x3, w1b[...], c1b[...]), 0.0)
    hb = jnp.maximum(mm(hb, w2b[...], c2b[...]), 0.0)
    hb = jnp.maximum(mm(hb, w3b[...], c3b[...]), 0.0)
    f2_ref[...] = mm(hb, wf2[...], cf2[...])
    bt = x.shape[0]
    for jb in range(_DU):
        r2 = mm(hb[:, 16 * jb:16 * (jb + 1)], wg2[jb], cg2[jb])
        g2_ref[:, 6 * jb:6 * (jb + 1), :] = \
            r2.reshape(bt, _DZ, 128)[:, :, :_DZU]


def kernel(x, params1, params2):
    B = x.shape[0]
    bt = _BT if B % _BT == 0 else B
    grid = (B // bt,)

    wargs = _build_weights(params1, params2)
    wargs = tuple(w.astype(jnp.bfloat16) if k % 2 == 0 else w
                  for k, w in enumerate(wargs))

    x_spec = pl.BlockSpec((bt, _DU), lambda i: (i, 0))
    w_specs = [pl.BlockSpec(w.shape, (lambda i: (0, 0)) if w.ndim == 2
               else (lambda i: (0, 0, 0))) for w in wargs]
    out_specs = [
        pl.BlockSpec((bt, _DU), lambda i: (i, 0)),
        pl.BlockSpec((bt, _DU, _DZU), lambda i: (i, 0, 0)),
        pl.BlockSpec((bt, _DZU), lambda i: (i, 0)),
        pl.BlockSpec((bt, _DZU, _DZU), lambda i: (i, 0, 0)),
    ]
    out_shape = [
        jax.ShapeDtypeStruct((B, _DU), jnp.float32),
        jax.ShapeDtypeStruct((B, _DU, _DZU), jnp.float32),
        jax.ShapeDtypeStruct((B, _DZU), jnp.float32),
        jax.ShapeDtypeStruct((B, _DZU, _DZU), jnp.float32),
    ]

    with jax.default_matmul_precision("bfloat16"):
        f1, g1, f2, g2 = pl.pallas_call(
            _body,
            grid=grid,
            in_specs=[x_spec] + list(w_specs),
            out_specs=out_specs,
            out_shape=out_shape,
            compiler_params=pltpu.CompilerParams(
                dimension_semantics=("parallel",),
                vmem_limit_bytes=100 * 1024 * 1024,
            ),
        )(x, *wargs)

    return (f1[..., None], g1, f2[..., None], g2)


# speedup vs baseline: 1.0342x; 1.0342x over previous
import jax
import jax.numpy as jnp
import numpy as np
from jax.experimental import pallas as pl
from jax.experimental.pallas import tpu as pltpu

_DU = 20
_DZ = 6
_DZU = _DU * _DZ
_BT = 128
_HI = jax.lax.Precision.HIGHEST
_MED = jax.lax.Precision.HIGHEST


def _g1_idx():
    idx = np.full((_DU, 128), 1 + 3 * _DZ, dtype=np.int32)
    for i in range(_DU):
        for t in range(3 * _DZ):
            idx[i, (6 * (i - 1) + t) % _DZU] = 1 + t
    return idx


def _g2_idx():
    idx = np.full((_DU, _DZ * 128), _DZ + 5 * _DZ * _DZ, dtype=np.int32)
    for jb in range(_DU):
        for s in range(_DZ):
            for t in range(5 * _DZ):
                idx[jb, 128 * s + (6 * (jb - 2) + t) % _DZU] = \
                    _DZ + 30 * s + t
    return idx


_G1_IDX = _g1_idx()
_G2_IDX = _g2_idx()


def _block_diag(blocks):
    g, k, n = blocks.shape
    eye = jnp.eye(g, dtype=blocks.dtype)
    bd = blocks[:, :, None, :] * eye[:, None, :, None]
    return bd.reshape(g * k, g * n)


def _expand_mid(W, b):
    wt = W.T[None].astype(jnp.float32)
    blocks = jnp.broadcast_to(wt, (_DU,) + wt.shape[1:])
    return _block_diag(blocks), jnp.tile(b, (_DU,))[None, :]


def _build_weights(params1, params2):
    (W1a, b1a), (W2a, b2a), (W3a, b3a), (W4a, b4a) = params1
    (W1b, b1b), (W2b, b2b), (W3b, b3b), (W4b, b4b) = params2

    def first(W1, b1):
        blocks = jnp.broadcast_to(W1.T[None], (_DU, 3, 16))
        eye = jnp.eye(_DU, dtype=jnp.float32)
        bd = blocks[:, :, None, :] * eye[:, None, :, None]
        w = bd.transpose(1, 0, 2, 3).reshape(3 * _DU, 16 * _DU)
        return w, jnp.tile(b1, (_DU,))[None, :]

    W1Pa, B1a = first(W1a, b1a)
    W1Pb, B1b = first(W1b, b1b)
    W2Pa, B2a = _expand_mid(W2a, b2a)
    W2Pb, B2b = _expand_mid(W2b, b2b)
    W3Pa, B3a = _expand_mid(W3a, b3a)
    W3Pb, B3b = _expand_mid(W3b, b3b)

    f1_blocks = jnp.broadcast_to(W4a[0][None, :, None], (_DU, 16, 1))
    Wf1 = _block_diag(f1_blocks)
    Bf1 = jnp.tile(b4a[0][None], (_DU,))[None, :]

    f2_blocks = jnp.broadcast_to(W4b[:_DZ].T[None], (_DU, 16, _DZ))
    Wf2 = _block_diag(f2_blocks)
    Bf2 = jnp.tile(b4b[:_DZ], (_DU,))[None, :]

    w4a_pad = jnp.concatenate([W4a.T, jnp.zeros((16, 1), jnp.float32)], 1)
    b4a_pad = jnp.concatenate([b4a, jnp.zeros((1,), jnp.float32)])
    Wg1 = w4a_pad[:, _G1_IDX].transpose(1, 0, 2)
    Bg1 = b4a_pad[_G1_IDX][:, None, :]

    w4b_pad = jnp.concatenate([W4b.T, jnp.zeros((16, 1), jnp.float32)], 1)
    b4b_pad = jnp.concatenate([b4b, jnp.zeros((1,), jnp.float32)])
    Wg2 = w4b_pad[:, _G2_IDX].transpose(1, 0, 2)
    Bg2 = b4b_pad[_G2_IDX][:, None, :]

    return (W1Pa, B1a, W2Pa, B2a, W3Pa, B3a, Wf1, Bf1, Wg1, Bg1,
            W1Pb, B1b, W2Pb, B2b, W3Pb, B3b, Wf2, Bf2, Wg2, Bg2)


def _body(x_ref,
          w1a, c1a, w2a, c2a, w3a, c3a, wf1, cf1, wg1, cg1,
          w1b, c1b, w2b, c2b, w3b, c3b, wf2, cf2, wg2, cg2,
          f1_ref, g1_ref, f2_ref, g2_ref):
    x = x_ref[...]
    xm = jnp.concatenate([x[:, -1:], x[:, :-1]], axis=1)
    xp = jnp.concatenate([x[:, 1:], x[:, :1]], axis=1)
    x3 = jnp.concatenate([xm, x, xp], axis=1)

    def mm(a, w, c):
        return jnp.dot(a.astype(jnp.bfloat16), w,
                       precision=jax.lax.Precision.DEFAULT,
                       preferred_element_type=jnp.float32) + c

    ha = jnp.maximum(mm(x3, w1a[...], c1a[...]), 0.0)
    ha = jnp.maximum(mm(ha, w2a[...], c2a[...]), 0.0)
    ha = jnp.maximum(mm(ha, w3a[...], c3a[...]), 0.0)
    f1_ref[...] = mm(ha, wf1[...], cf1[...])
    bt = x.shape[0]
    for q, n in ((0, 8), (8, 8), (16, 4)):
        grp = jnp.concatenate(
            [mm(ha[:, 16 * (q + j):16 * (q + j + 1)], wg1[q + j],
                cg1[q + j]) for j in range(n)], axis=1)
        g1_ref[:, q:q + n, :] = grp.reshape(bt, n, 128)[:, :, :_DZU]

    hb = jnp.maximum(mm(x3, w1b[...], c1b[...]), 0.0)
    hb = jnp.maximum(mm(hb, w2b[...], c2b[...]), 0.0)
    hb = jnp.maximum(mm(hb, w3b[...], c3b[...]), 0.0)
    f2_ref[...] = mm(hb, wf2[...], cf2[...])
    for q in range(0, _DU, 4):
        grp = jnp.concatenate(
            [mm(hb[:, 16 * (q + j):16 * (q + j + 1)], wg2[q + j],
                cg2[q + j]) for j in range(4)], axis=1)
        g2_ref[:, 6 * q:6 * q + 24, :] = \
            grp.reshape(bt, 24, 128)[:, :, :_DZU]


def kernel(x, params1, params2):
    B = x.shape[0]
    bt = _BT if B % _BT == 0 else B
    grid = (B // bt,)

    wargs = _build_weights(params1, params2)
    wargs = tuple(w.astype(jnp.bfloat16) if k % 2 == 0 else w
                  for k, w in enumerate(wargs))

    x_spec = pl.BlockSpec((bt, _DU), lambda i: (i, 0))
    w_specs = [pl.BlockSpec(w.shape, (lambda i: (0, 0)) if w.ndim == 2
               else (lambda i: (0, 0, 0))) for w in wargs]
    out_specs = [
        pl.BlockSpec((bt, _DU), lambda i: (i, 0)),
        pl.BlockSpec((bt, _DU, _DZU), lambda i: (i, 0, 0)),
        pl.BlockSpec((bt, _DZU), lambda i: (i, 0)),
        pl.BlockSpec((bt, _DZU, _DZU), lambda i: (i, 0, 0)),
    ]
    out_shape = [
        jax.ShapeDtypeStruct((B, _DU), jnp.float32),
        jax.ShapeDtypeStruct((B, _DU, _DZU), jnp.float32),
        jax.ShapeDtypeStruct((B, _DZU), jnp.float32),
        jax.ShapeDtypeStruct((B, _DZU, _DZU), jnp.float32),
    ]

    with jax.default_matmul_precision("bfloat16"):
        f1, g1, f2, g2 = pl.pallas_call(
            _body,
            grid=grid,
            in_specs=[x_spec] + list(w_specs),
            out_specs=out_specs,
            out_shape=out_shape,
            compiler_params=pltpu.CompilerParams(
                dimension_semantics=("parallel",),
                vmem_limit_bytes=100 * 1024 * 1024,
            ),
        )(x, *wargs)

    return (f1[..., None], g1, f2[..., None], g2)


# speedup vs baseline: 1.3087x; 1.2654x over previous
import jax
import jax.numpy as jnp
import numpy as np
from jax.experimental import pallas as pl
from jax.experimental.pallas import tpu as pltpu

_DU = 20
_DZ = 6
_DZU = _DU * _DZ
_BT = 128
_HI = jax.lax.Precision.HIGHEST
_MED = jax.lax.Precision.HIGHEST


def _g1_idx():
    idx = np.full((_DU, _DZU), 1 + 3 * _DZ, dtype=np.int32)
    for i in range(_DU):
        for t in range(3 * _DZ):
            idx[i, (6 * (i - 1) + t) % _DZU] = 1 + t
    return idx


def _g2_idx():
    idx = np.full((_DU, _DZ * _DZU), _DZ + 5 * _DZ * _DZ, dtype=np.int32)
    for jb in range(_DU):
        for s in range(_DZ):
            for t in range(5 * _DZ):
                idx[jb, 120 * s + (6 * (jb - 2) + t) % _DZU] = \
                    _DZ + 30 * s + t
    return idx


_G1_IDX = _g1_idx()
_G2_IDX = _g2_idx()


def _block_diag(blocks):
    g, k, n = blocks.shape
    eye = jnp.eye(g, dtype=blocks.dtype)
    bd = blocks[:, :, None, :] * eye[:, None, :, None]
    return bd.reshape(g * k, g * n)


def _expand_mid(W, b):
    wt = W.T[None].astype(jnp.float32)
    blocks = jnp.broadcast_to(wt, (_DU,) + wt.shape[1:])
    return _block_diag(blocks), jnp.tile(b, (_DU,))[None, :]


def _build_weights(params1, params2):
    (W1a, b1a), (W2a, b2a), (W3a, b3a), (W4a, b4a) = params1
    (W1b, b1b), (W2b, b2b), (W3b, b3b), (W4b, b4b) = params2

    def first(W1, b1):
        blocks = jnp.broadcast_to(W1.T[None], (_DU, 3, 16))
        eye = jnp.eye(_DU, dtype=jnp.float32)
        bd = blocks[:, :, None, :] * eye[:, None, :, None]
        w = bd.transpose(1, 0, 2, 3).reshape(3 * _DU, 16 * _DU)
        return w, jnp.tile(b1, (_DU,))[None, :]

    W1Pa, B1a = first(W1a, b1a)
    W1Pb, B1b = first(W1b, b1b)
    W2Pa, B2a = _expand_mid(W2a, b2a)
    W2Pb, B2b = _expand_mid(W2b, b2b)
    W3Pa, B3a = _expand_mid(W3a, b3a)
    W3Pb, B3b = _expand_mid(W3b, b3b)

    f1_blocks = jnp.broadcast_to(W4a[0][None, :, None], (_DU, 16, 1))
    Wf1 = _block_diag(f1_blocks)
    Bf1 = jnp.tile(b4a[0][None], (_DU,))[None, :]

    f2_blocks = jnp.broadcast_to(W4b[:_DZ].T[None], (_DU, 16, _DZ))
    Wf2 = _block_diag(f2_blocks)
    Bf2 = jnp.tile(b4b[:_DZ], (_DU,))[None, :]

    w4a_pad = jnp.concatenate([W4a.T, jnp.zeros((16, 1), jnp.float32)], 1)
    b4a_pad = jnp.concatenate([b4a, jnp.zeros((1,), jnp.float32)])
    Wg1 = w4a_pad[:, _G1_IDX].transpose(1, 0, 2)
    Bg1 = b4a_pad[_G1_IDX][:, None, :]

    w4b_pad = jnp.concatenate([W4b.T, jnp.zeros((16, 1), jnp.float32)], 1)
    b4b_pad = jnp.concatenate([b4b, jnp.zeros((1,), jnp.float32)])
    Wg2 = w4b_pad[:, _G2_IDX].transpose(1, 0, 2)
    Bg2 = b4b_pad[_G2_IDX][:, None, :]

    return (W1Pa, B1a, W2Pa, B2a, W3Pa, B3a, Wf1, Bf1, Wg1, Bg1,
            W1Pb, B1b, W2Pb, B2b, W3Pb, B3b, Wf2, Bf2, Wg2, Bg2)


def _body(x_ref,
          w1a, c1a, w2a, c2a, w3a, c3a, wf1, cf1, wg1, cg1,
          w1b, c1b, w2b, c2b, w3b, c3b, wf2, cf2, wg2, cg2,
          f1_ref, g1_ref, f2_ref, g2_ref):
    x = x_ref[...]
    xm = jnp.concatenate([x[:, -1:], x[:, :-1]], axis=1)
    xp = jnp.concatenate([x[:, 1:], x[:, :1]], axis=1)
    x3 = jnp.concatenate([xm, x, xp], axis=1)

    def mm(a, w, c):
        return jnp.dot(a.astype(jnp.bfloat16), w,
                       precision=jax.lax.Precision.DEFAULT,
                       preferred_element_type=jnp.float32) + c

    ha = jnp.maximum(mm(x3, w1a[...], c1a[...]), 0.0)
    ha = jnp.maximum(mm(ha, w2a[...], c2a[...]), 0.0)
    ha = jnp.maximum(mm(ha, w3a[...], c3a[...]), 0.0)
    f1_ref[...] = mm(ha, wf1[...], cf1[...])
    for i in range(_DU):
        g1_ref[:, _DZU * i:_DZU * (i + 1)] = mm(
            ha[:, 16 * i:16 * (i + 1)], wg1[i], cg1[i])

    hb = jnp.maximum(mm(x3, w1b[...], c1b[...]), 0.0)
    hb = jnp.maximum(mm(hb, w2b[...], c2b[...]), 0.0)
    hb = jnp.maximum(mm(hb, w3b[...], c3b[...]), 0.0)
    f2_ref[...] = mm(hb, wf2[...], cf2[...])
    for jb in range(_DU):
        g2_ref[:, 6 * _DZU * jb:6 * _DZU * (jb + 1)] = mm(
            hb[:, 16 * jb:16 * (jb + 1)], wg2[jb], cg2[jb])


def kernel(x, params1, params2):
    B = x.shape[0]
    bt = _BT if B % _BT == 0 else B
    grid = (B // bt,)

    wargs = _build_weights(params1, params2)
    wargs = tuple(w.astype(jnp.bfloat16) if k % 2 == 0 else w
                  for k, w in enumerate(wargs))

    x_spec = pl.BlockSpec((bt, _DU), lambda i: (i, 0))
    w_specs = [pl.BlockSpec(w.shape, (lambda i: (0, 0)) if w.ndim == 2
               else (lambda i: (0, 0, 0))) for w in wargs]
    out_specs = [
        pl.BlockSpec((bt, _DU), lambda i: (i, 0)),
        pl.BlockSpec((bt, _DU * _DZU), lambda i: (i, 0)),
        pl.BlockSpec((bt, _DZU), lambda i: (i, 0)),
        pl.BlockSpec((bt, _DZU * _DZU), lambda i: (i, 0)),
    ]
    out_shape = [
        jax.ShapeDtypeStruct((B, _DU), jnp.float32),
        jax.ShapeDtypeStruct((B, _DU * _DZU), jnp.float32),
        jax.ShapeDtypeStruct((B, _DZU), jnp.float32),
        jax.ShapeDtypeStruct((B, _DZU * _DZU), jnp.float32),
    ]

    with jax.default_matmul_precision("bfloat16"):
        f1, g1, f2, g2 = pl.pallas_call(
            _body,
            grid=grid,
            in_specs=[x_spec] + list(w_specs),
            out_specs=out_specs,
            out_shape=out_shape,
            compiler_params=pltpu.CompilerParams(
                dimension_semantics=("parallel",),
                vmem_limit_bytes=100 * 1024 * 1024,
            ),
        )(x, *wargs)

    return (f1[..., None], g1.reshape(B, _DU, _DZU),
            f2[..., None], g2.reshape(B, _DZU, _DZU))
